# initial kernel scaffold (unmeasured)
import jax
import jax.numpy as jnp
from jax import lax
from jax.experimental import pallas as pl
from jax.experimental.pallas import tpu as pltpu

N_DEV = 32


def kernel(x, Win0, Wout0, Win1, Wout1, Win2, Wout2):
    B, D = x.shape
    rows = B // N_DEV

    def body(x_ref, win0, wout0, win1, wout1, win2, wout2, out_ref,
             xg_ref, p_ref, red_ref, rs_ref, send_sems, p1_sems, p2_sems):
        me = lax.axis_index("i")

        barrier = pltpu.get_barrier_semaphore()
        for d in range(1, N_DEV):
            pl.semaphore_signal(
                barrier, inc=1,
                device_id=((me + d) % N_DEV,),
                device_id_type=pl.DeviceIdType.MESH,
            )
        pl.semaphore_wait(barrier, N_DEV - 1)

        def layer(xb, win, wout, last):
            h = jnp.dot(xb, win[:], preferred_element_type=jnp.float32)
            h = jnp.maximum(h, 0.0).astype(jnp.bfloat16)
            p_ref[:] = jnp.dot(h, wout[:], preferred_element_type=jnp.float32)

            sends = []
            for d in range(1, N_DEV):
                dst = (me + d) % N_DEV
                rdma = pltpu.make_async_remote_copy(
                    src_ref=p_ref.at[pl.ds(dst * rows, rows)],
                    dst_ref=rs_ref.at[N_DEV - d],
                    send_sem=send_sems.at[d - 1],
                    recv_sem=p1_sems.at[N_DEV - d],
                    device_id=(dst,),
                    device_id_type=pl.DeviceIdType.MESH,
                )
                rdma.start()
                sends.append(rdma)
            rs_ref[0] = p_ref[pl.ds(me * rows, rows)]
            for r in sends:
                r.wait_recv()
            red = jnp.sum(rs_ref[:], axis=0)
            for r in sends:
                r.wait_send()

            if last:
                out_ref[:] = red
                return None

            red_ref[:] = red
            xg_ref[pl.ds(me * rows, rows)] = red
            sends2 = []
            for d in range(1, N_DEV):
                dst = (me + d) % N_DEV
                rdma = pltpu.make_async_remote_copy(
                    src_ref=red_ref,
                    dst_ref=xg_ref.at[pl.ds(me * rows, rows)],
                    send_sem=send_sems.at[d - 1],
                    recv_sem=p2_sems.at[N_DEV - d],
                    device_id=(dst,),
                    device_id_type=pl.DeviceIdType.MESH,
                )
                rdma.start()
                sends2.append(rdma)
            for r in sends2:
                r.wait_recv()
            for r in sends2:
                r.wait_send()
            return xg_ref[:].astype(jnp.bfloat16)

        xb = x_ref[:].astype(jnp.bfloat16)
        xb = layer(xb, win0, wout0, last=False)
        xb = layer(xb, win1, wout1, last=False)
        layer(xb, win2, wout2, last=True)

    return pl.pallas_call(
        body,
        out_shape=jax.ShapeDtypeStruct((rows, D), jnp.float32),
        in_specs=[pl.BlockSpec(memory_space=pltpu.VMEM)] * 7,
        out_specs=pl.BlockSpec(memory_space=pltpu.VMEM),
        scratch_shapes=[
            pltpu.VMEM((B, D), jnp.float32),
            pltpu.VMEM((B, D), jnp.float32),
            pltpu.VMEM((rows, D), jnp.float32),
            pltpu.VMEM((N_DEV, rows, D), jnp.float32),
            pltpu.SemaphoreType.DMA((N_DEV - 1,)),
            pltpu.SemaphoreType.DMA((N_DEV,)),
            pltpu.SemaphoreType.DMA((N_DEV,)),
        ],
        compiler_params=pltpu.CompilerParams(collective_id=0),
    )(x, Win0, Wout0, Win1, Wout1, Win2, Wout2)


# baseline (device time: 45766 ns/iter reference)
import jax
import jax.numpy as jnp
from jax import lax
from jax.experimental import pallas as pl
from jax.experimental.pallas import tpu as pltpu

N_DEV = 32


def kernel(x, Win0, Wout0, Win1, Wout1, Win2, Wout2):
    B, D = x.shape
    rows = B // N_DEV

    def body(x_ref, win0, wout0, win1, wout1, win2, wout2, out_ref,
             xg_ref, p_ref, red_ref, rs_ref, send_sems, p1_sems, p2_sems,
             local_sem):
        me = lax.axis_index("i")

        barrier = pltpu.get_barrier_semaphore()
        for d in range(1, N_DEV):
            pl.semaphore_signal(
                barrier, inc=1,
                device_id=((me + d) % N_DEV,),
                device_id_type=pl.DeviceIdType.MESH,
            )
        pl.semaphore_wait(barrier, N_DEV - 1)

        def layer(xb, win, wout, last):
            h = jnp.dot(xb, win[:], preferred_element_type=jnp.float32)
            h = jnp.maximum(h, 0.0).astype(jnp.bfloat16)
            p_ref[:] = jnp.dot(h, wout[:], preferred_element_type=jnp.float32)

            sends = []
            for d in range(1, N_DEV):
                dst = (me + d) % N_DEV
                rdma = pltpu.make_async_remote_copy(
                    src_ref=p_ref.at[pl.ds(dst * rows, rows)],
                    dst_ref=rs_ref.at[N_DEV - d],
                    send_sem=send_sems.at[d - 1],
                    recv_sem=p1_sems.at[N_DEV - d],
                    device_id=(dst,),
                    device_id_type=pl.DeviceIdType.MESH,
                )
                rdma.start()
                sends.append(rdma)
            own = pltpu.make_async_copy(
                p_ref.at[pl.ds(me * rows, rows)], rs_ref.at[0], local_sem)
            own.start()
            own.wait()
            for r in sends:
                r.wait_recv()
            red = jnp.sum(rs_ref[:], axis=0)
            for r in sends:
                r.wait_send()

            if last:
                out_ref[:] = red
                return None

            red_ref[:] = red
            own2 = pltpu.make_async_copy(
                red_ref, xg_ref.at[pl.ds(me * rows, rows)], local_sem)
            own2.start()
            sends2 = []
            for d in range(1, N_DEV):
                dst = (me + d) % N_DEV
                rdma = pltpu.make_async_remote_copy(
                    src_ref=red_ref,
                    dst_ref=xg_ref.at[pl.ds(me * rows, rows)],
                    send_sem=send_sems.at[d - 1],
                    recv_sem=p2_sems.at[N_DEV - d],
                    device_id=(dst,),
                    device_id_type=pl.DeviceIdType.MESH,
                )
                rdma.start()
                sends2.append(rdma)
            own2.wait()
            for r in sends2:
                r.wait_recv()
            for r in sends2:
                r.wait_send()
            return xg_ref[:].astype(jnp.bfloat16)

        xb = x_ref[:].astype(jnp.bfloat16)
        xb = layer(xb, win0, wout0, last=False)
        xb = layer(xb, win1, wout1, last=False)
        layer(xb, win2, wout2, last=True)

    return pl.pallas_call(
        body,
        out_shape=jax.ShapeDtypeStruct((rows, D), jnp.float32),
        in_specs=[pl.BlockSpec(memory_space=pltpu.VMEM)] * 7,
        out_specs=pl.BlockSpec(memory_space=pltpu.VMEM),
        scratch_shapes=[
            pltpu.VMEM((B, D), jnp.float32),
            pltpu.VMEM((B, D), jnp.float32),
            pltpu.VMEM((rows, D), jnp.float32),
            pltpu.VMEM((N_DEV, rows, D), jnp.float32),
            pltpu.SemaphoreType.DMA((N_DEV - 1,)),
            pltpu.SemaphoreType.DMA((N_DEV,)),
            pltpu.SemaphoreType.DMA((N_DEV,)),
            pltpu.SemaphoreType.DMA,
        ],
        compiler_params=pltpu.CompilerParams(collective_id=0),
    )(x, Win0, Wout0, Win1, Wout1, Win2, Wout2)


# device time: 43656 ns/iter; 1.0483x vs baseline; 1.0483x over previous
import jax
import jax.numpy as jnp
from jax import lax
from jax.experimental import pallas as pl
from jax.experimental.pallas import tpu as pltpu

N_DEV = 32


def kernel(x, Win0, Wout0, Win1, Wout1, Win2, Wout2):
    B, D = x.shape
    rows = B // N_DEV

    def body(x_ref, win0, wout0, win1, wout1, win2, wout2, out_ref,
             xg_ref, p_ref, red_ref, rs_ref, send_sems, p1_sems, p2_sems,
             local_sem):
        me = lax.axis_index("i")

        barrier = pltpu.get_barrier_semaphore()
        for d in range(1, N_DEV):
            pl.semaphore_signal(
                barrier, inc=1,
                device_id=((me + d) % N_DEV,),
                device_id_type=pl.DeviceIdType.MESH,
            )
        pl.semaphore_wait(barrier, N_DEV - 1)

        def layer(xb, win, wout, last):
            h = jnp.dot(xb, win[:], preferred_element_type=jnp.float32)
            h = jnp.maximum(h, 0.0).astype(jnp.bfloat16)
            p_ref[:] = jnp.dot(
                h, wout[:], preferred_element_type=jnp.float32
            ).astype(jnp.bfloat16)

            sends = []
            for d in range(1, N_DEV):
                dst = (me + d) % N_DEV
                rdma = pltpu.make_async_remote_copy(
                    src_ref=p_ref.at[pl.ds(dst * rows, rows)],
                    dst_ref=rs_ref.at[N_DEV - d],
                    send_sem=send_sems.at[d - 1],
                    recv_sem=p1_sems.at[N_DEV - d],
                    device_id=(dst,),
                    device_id_type=pl.DeviceIdType.MESH,
                )
                rdma.start()
                sends.append(rdma)
            own = pltpu.make_async_copy(
                p_ref.at[pl.ds(me * rows, rows)], rs_ref.at[0], local_sem)
            own.start()
            own.wait()
            for r in sends:
                r.wait_recv()
            red = jnp.sum(rs_ref[:].astype(jnp.float32), axis=0)
            for r in sends:
                r.wait_send()

            if last:
                out_ref[:] = red
                return None

            red_ref[:] = red.astype(jnp.bfloat16)
            own2 = pltpu.make_async_copy(
                red_ref, xg_ref.at[pl.ds(me * rows, rows)], local_sem)
            own2.start()
            sends2 = []
            for d in range(1, N_DEV):
                dst = (me + d) % N_DEV
                rdma = pltpu.make_async_remote_copy(
                    src_ref=red_ref,
                    dst_ref=xg_ref.at[pl.ds(me * rows, rows)],
                    send_sem=send_sems.at[d - 1],
                    recv_sem=p2_sems.at[N_DEV - d],
                    device_id=(dst,),
                    device_id_type=pl.DeviceIdType.MESH,
                )
                rdma.start()
                sends2.append(rdma)
            own2.wait()
            for r in sends2:
                r.wait_recv()
            for r in sends2:
                r.wait_send()
            return xg_ref[:]

        xb = x_ref[:].astype(jnp.bfloat16)
        xb = layer(xb, win0, wout0, last=False)
        xb = layer(xb, win1, wout1, last=False)
        layer(xb, win2, wout2, last=True)

    return pl.pallas_call(
        body,
        out_shape=jax.ShapeDtypeStruct((rows, D), jnp.float32),
        in_specs=[pl.BlockSpec(memory_space=pltpu.VMEM)] * 7,
        out_specs=pl.BlockSpec(memory_space=pltpu.VMEM),
        scratch_shapes=[
            pltpu.VMEM((B, D), jnp.bfloat16),
            pltpu.VMEM((B, D), jnp.bfloat16),
            pltpu.VMEM((rows, D), jnp.bfloat16),
            pltpu.VMEM((N_DEV, rows, D), jnp.bfloat16),
            pltpu.SemaphoreType.DMA((N_DEV - 1,)),
            pltpu.SemaphoreType.DMA((N_DEV,)),
            pltpu.SemaphoreType.DMA((N_DEV,)),
            pltpu.SemaphoreType.DMA,
        ],
        compiler_params=pltpu.CompilerParams(collective_id=0),
    )(x, Win0, Wout0, Win1, Wout1, Win2, Wout2)
